# baseline (speedup 1.0000x reference)
import functools
import math

import jax
import jax.numpy as jnp
from jax import lax
from jax.experimental import pallas as pl
from jax.experimental.pallas import tpu as pltpu
from jax.experimental.pallas import tpu_sc as plsc

NUM_EMBEDDINGS = 1000000
D_MODEL = 64
BATCH = 4096
SEQ = 50
SCALE = math.sqrt(D_MODEL)

B_TOTAL = BATCH * SEQ
NC = 2
NS = 16
NW = NC * NS
B_PER_W = B_TOTAL // NW
CHUNK = 128
NCHUNK = B_PER_W // CHUNK
NBUF = 2
ROWS_PER_STEP = 4
VECS_PER_ROW = D_MODEL // 16


def _sc_embed(idx_flat, table):
    mesh = plsc.VectorSubcoreMesh(core_axis_name="c", subcore_axis_name="s")

    @functools.partial(
        pl.kernel,
        mesh=mesh,
        compiler_params=pltpu.CompilerParams(use_tc_tiling_on_sc=False),
        out_type=jax.ShapeDtypeStruct((B_TOTAL, D_MODEL), jnp.float32),
        scratch_types=(
            [pltpu.VMEM((B_PER_W,), jnp.int32)]
            + [pltpu.VMEM((CHUNK, D_MODEL), jnp.float32) for _ in range(NBUF)]
            + [pltpu.SemaphoreType.DMA for _ in range(NBUF)]
        ),
    )
    def k(idx_hbm, table_hbm, out_hbm, idx_v, *bufs_and_sems):
        rows = bufs_and_sems[:NBUF]
        sems = bufs_and_sems[NBUF:]
        wid = lax.axis_index("s") * NC + lax.axis_index("c")
        base = wid * B_PER_W

        pltpu.sync_copy(idx_hbm.at[pl.ds(base, B_PER_W)], idx_v)

        def idx_chunk(g):
            return idx_v.at[pl.ds(g * CHUNK, CHUNK)]

        for b in range(NBUF):
            pltpu.async_copy(table_hbm.at[idx_chunk(b)], rows[b], sems[b])

        def scale_rows(buf):
            def body(r, _):
                for rr in range(ROWS_PER_STEP):
                    for j in range(VECS_PER_ROW):
                        sl = (r * ROWS_PER_STEP + rr, pl.ds(16 * j, 16))
                        buf[sl] = buf[sl] * SCALE
                return 0
            lax.fori_loop(0, CHUNK // ROWS_PER_STEP, body, 0, unroll=False)

        def outer(i, _):
            for b in range(NBUF):
                g = i * NBUF + b
                pltpu.make_async_copy(table_hbm.at[idx_chunk(g)], rows[b],
                                      sems[b]).wait()
                scale_rows(rows[b])
                pltpu.sync_copy(rows[b],
                                out_hbm.at[pl.ds(base + g * CHUNK, CHUNK)])
                nxt = g + NBUF

                @pl.when(nxt < NCHUNK)
                def _():
                    pltpu.async_copy(table_hbm.at[idx_chunk(nxt)], rows[b],
                                     sems[b])
            return 0

        lax.fori_loop(0, NCHUNK // NBUF, outer, 0, unroll=False)

    return k(idx_flat, table)


@jax.jit
def kernel(input, W):
    idx_flat = input.reshape(B_TOTAL)
    out = _sc_embed(idx_flat, W)
    return out.reshape(BATCH, SEQ, D_MODEL)
